# baseline (speedup 1.0000x reference)
import functools

import jax
import jax.numpy as jnp
from jax import lax
from jax.experimental import pallas as pl
from jax.experimental.pallas import tpu as pltpu
from jax.experimental.pallas import tpu_sc as plsc

NUM_CORES = 2
NUM_SUBCORES = 16
LANES = 16
NW = NUM_CORES * NUM_SUBCORES

B = 4096
D = 64
V = 100000
TILES = (V + 127) // 128
BPW = B // NW
GROUPS = BPW // LANES

CAP = 256
HCAP = CAP // 2
NPAIR = 13
SB = B + 128
DUMP = B + 64


def _route_body(uids_hbm, iids_hbm, uet_hbm, iet_hbm, su_hbm, si_hbm,
                uids_v, iids_v, list_ids, list_pos, posflat, posidx,
                cols_t, tilebuf, extbuf, sem0, sem1, ssem):
    wid = lax.axis_index("s") * NUM_CORES + lax.axis_index("c")
    lo = wid * 24 + jnp.minimum(wid, 14)

    pltpu.sync_copy(uids_hbm, uids_v)
    pltpu.sync_copy(iids_hbm, iids_v)

    lane = jnp.arange(LANES, dtype=jnp.int32)
    dump = jnp.full((LANES,), DUMP, dtype=jnp.int32)
    sems = (sem0, sem1)
    last = TILES - 1

    for ids_v, tbl_hbm, staged_hbm in ((uids_v, uet_hbm, su_hbm),
                                       (iids_v, iet_hbm, si_hbm)):
        def slc(tg, tbl_hbm=tbl_hbm):
            tc = jnp.minimum(tg, last)
            return tbl_hbm.at[:, pl.ds(pl.multiple_of(tc * 128, 128), 128)]

        def fchunk(c, e):
            v = ids_v[pl.ds(c * LANES, LANES)]
            t = (v >> 7) - lo
            mine = (t >= 0) & (t < jnp.where(wid < 14, 25, 24))
            plsc.store_compressed(list_ids.at[pl.ds(e, LANES)], v, mask=mine)
            plsc.store_compressed(list_pos.at[pl.ds(e, LANES)],
                                  c * LANES + lane, mask=mine)
            return e + plsc.all_reduce_population_count(mine)[0]

        cnt = lax.fori_loop(0, B // LANES, fchunk, 0)

        def initp(c, _):
            posflat[pl.ds(c * LANES, LANES)] = dump
            return 0

        lax.fori_loop(0, CAP // LANES, initp, 0)

        pltpu.async_copy(slc(lo), tilebuf.at[0], sem0)
        pltpu.async_copy(slc(lo + 1), tilebuf.at[1], sem1)

        def pair_body(j, e_base):
            for sub in range(2):
                t = 2 * j + sub
                tg = lo + t
                buf = tilebuf.at[sub]
                pltpu.make_async_copy(slc(tg), buf, sems[sub]).wait()

                def schunk(c2, et, tg=tg):
                    v2 = list_ids[pl.ds(c2 * LANES, LANES)]
                    p2 = list_pos[pl.ds(c2 * LANES, LANES)]
                    valid = (c2 * LANES + lane) < cnt
                    m2 = ((v2 >> 7) == tg) & valid
                    plsc.store_compressed(cols_t.at[pl.ds(et, LANES)],
                                          v2 & 127, mask=m2)
                    plsc.store_compressed(
                        posflat.at[pl.ds(e_base + et, LANES)], p2, mask=m2)
                    return et + plsc.all_reduce_population_count(m2)[0]

                nchunk = (cnt + LANES - 1) // LANES
                cnt_t = lax.fori_loop(0, nchunk, schunk, 0)

                for cc in range(2):
                    cvec = cols_t[pl.ds(cc * LANES, LANES)]
                    for k in range(LANES):
                        idx_e = cc * LANES + k

                        @pl.when(idx_e < cnt_t)
                        def _(cvec=cvec, k=k, idx_e=idx_e, buf=buf,
                              e_base=e_base):
                            csp = jnp.full((LANES,), cvec[k],
                                           dtype=jnp.int32)
                            rsp = jnp.full((LANES,), e_base + idx_e,
                                           dtype=jnp.int32)
                            for q in range(D // LANES):
                                val = plsc.load_gather(
                                    buf, [q * LANES + lane, csp])
                                plsc.store_scatter(
                                    extbuf, [rsp, q * LANES + lane], val)

                e_base = e_base + cnt_t

                @pl.when(t + 2 < 2 * NPAIR)
                def _(tg=tg, buf=buf, sub=sub):
                    pltpu.async_copy(slc(tg + 2), buf, sems[sub])
            return e_base

        lax.fori_loop(0, NPAIR, pair_body, 0)

        for h in range(2):
            for c in range(HCAP // LANES):
                posidx[h, pl.ds(c * LANES, LANES)] = (
                    posflat[pl.ds(h * HCAP + c * LANES, LANES)])

        cps = [
            pltpu.async_copy(extbuf.at[pl.ds(h * HCAP, HCAP)],
                             staged_hbm.at[posidx.at[h]], ssem)
            for h in range(2)
        ]
        for cp in cps:
            cp.wait()


def _score_body(uids_hbm, iids_hbm, su_hbm, si_hbm, ubias_hbm, ibias_hbm,
                out_hbm, uid_v, iid_v, urows_v, irows_v, ub_v, ib_v, score_v,
                sem):
    wid = lax.axis_index("s") * NUM_CORES + lax.axis_index("c")
    base = wid * BPW

    pltpu.sync_copy(uids_hbm.at[pl.ds(base, BPW)], uid_v)
    pltpu.sync_copy(iids_hbm.at[pl.ds(base, BPW)], iid_v)

    cps = [
        pltpu.async_copy(su_hbm.at[pl.ds(base, BPW)], urows_v, sem),
        pltpu.async_copy(si_hbm.at[pl.ds(base, BPW)], irows_v, sem),
        pltpu.async_copy(ubias_hbm.at[0].at[uid_v], ub_v, sem),
        pltpu.async_copy(ibias_hbm.at[0].at[iid_v], ib_v, sem),
    ]
    for cp in cps:
        cp.wait()

    lane = jnp.arange(LANES, dtype=jnp.int32)
    rows = [g * LANES + lane for g in range(GROUPS)]
    accs0 = tuple(ub_v[pl.ds(g * LANES, LANES)] + ib_v[pl.ds(g * LANES, LANES)]
                  for g in range(GROUPS))

    def dstep(d, accs):
        col = jnp.full((LANES,), d, dtype=jnp.int32)
        return tuple(
            accs[g]
            + plsc.load_gather(urows_v, [rows[g], col])
            * plsc.load_gather(irows_v, [rows[g], col])
            for g in range(GROUPS)
        )

    accs = lax.fori_loop(0, D, dstep, accs0)
    for g in range(GROUPS):
        score_v[pl.ds(g * LANES, LANES)] = accs[g]

    pltpu.sync_copy(score_v, out_hbm.at[pl.ds(base, BPW)])


@jax.jit
def _svd_score(user_ids, item_ids, uet, iet, ubias_t, ibias_t):
    mesh = plsc.VectorSubcoreMesh(core_axis_name="c", subcore_axis_name="s")
    route = functools.partial(
        pl.kernel,
        out_type=(jax.ShapeDtypeStruct((SB, 128), jnp.float32),
                  jax.ShapeDtypeStruct((SB, 128), jnp.float32)),
        mesh=mesh,
        compiler_params=pltpu.CompilerParams(
            needs_layout_passes=False, use_tc_tiling_on_sc=True,
            disable_bounds_checks=True),
        scratch_types=[
            pltpu.VMEM((B,), jnp.int32),
            pltpu.VMEM((B,), jnp.int32),
            pltpu.VMEM((CAP + LANES,), jnp.int32),
            pltpu.VMEM((CAP + LANES,), jnp.int32),
            pltpu.VMEM((CAP + LANES,), jnp.int32),
            pltpu.VMEM((2, HCAP), jnp.int32),
            pltpu.VMEM((2 * LANES + LANES,), jnp.int32),
            pltpu.VMEM((2, D, 128), jnp.float32),
            pltpu.VMEM((CAP, 128), jnp.float32),
            pltpu.SemaphoreType.DMA,
            pltpu.SemaphoreType.DMA,
            pltpu.SemaphoreType.DMA,
        ],
    )(_route_body)
    staged_u, staged_i = route(user_ids, item_ids, uet, iet)

    score = functools.partial(
        pl.kernel,
        out_type=jax.ShapeDtypeStruct((B,), jnp.float32),
        mesh=mesh,
        compiler_params=pltpu.CompilerParams(
            needs_layout_passes=False, use_tc_tiling_on_sc=False),
        scratch_types=[
            pltpu.VMEM((BPW,), jnp.int32),
            pltpu.VMEM((BPW,), jnp.int32),
            pltpu.VMEM((BPW, 128), jnp.float32),
            pltpu.VMEM((BPW, 128), jnp.float32),
            pltpu.VMEM((BPW,), jnp.float32),
            pltpu.VMEM((BPW,), jnp.float32),
            pltpu.VMEM((BPW,), jnp.float32),
            pltpu.SemaphoreType.DMA,
        ],
    )(_score_body)
    return score(user_ids, item_ids, staged_u, staged_i, ubias_t, ibias_t)


def kernel(user_ids, item_ids, user_emb, item_emb, user_bias, item_bias,
           average_score):
    del average_score
    score = _svd_score(user_ids, item_ids, user_emb.T, item_emb.T,
                       user_bias.T, item_bias.T)
    return score.reshape(B, 1)


# speedup vs baseline: 2.6525x; 2.6525x over previous
import functools

import jax
import jax.numpy as jnp
from jax import lax
from jax.experimental import pallas as pl
from jax.experimental.pallas import tpu as pltpu
from jax.experimental.pallas import tpu_sc as plsc

NUM_CORES = 2
NUM_SUBCORES = 16
LANES = 16
NW = NUM_CORES * NUM_SUBCORES

B = 4096
D = 64
BPW = B // NW
GROUPS = BPW // LANES


def _svd_body(uids_hbm, iids_hbm, uemb_hbm, iemb_hbm, ubias_hbm, ibias_hbm,
              out_hbm, uid_v, iid_v, urows_v, irows_v, ub_v, ib_v, score_v,
              sem):
    wid = lax.axis_index("s") * NUM_CORES + lax.axis_index("c")
    base = wid * BPW

    pltpu.sync_copy(uids_hbm.at[pl.ds(base, BPW)], uid_v)
    pltpu.sync_copy(iids_hbm.at[pl.ds(base, BPW)], iid_v)

    cps = [
        pltpu.async_copy(uemb_hbm.at[uid_v], urows_v, sem),
        pltpu.async_copy(iemb_hbm.at[iid_v], irows_v, sem),
        pltpu.async_copy(ubias_hbm.at[0].at[uid_v], ub_v, sem),
        pltpu.async_copy(ibias_hbm.at[0].at[iid_v], ib_v, sem),
    ]
    for cp in cps:
        cp.wait()

    lane = jnp.arange(LANES, dtype=jnp.int32)
    rows = [g * LANES + lane for g in range(GROUPS)]
    accs0 = tuple(ub_v[pl.ds(g * LANES, LANES)] + ib_v[pl.ds(g * LANES, LANES)]
                  for g in range(GROUPS))

    def dstep(d, accs):
        col = jnp.full((LANES,), d, dtype=jnp.int32)
        return tuple(
            accs[g]
            + plsc.load_gather(urows_v, [rows[g], col])
            * plsc.load_gather(irows_v, [rows[g], col])
            for g in range(GROUPS)
        )

    accs = lax.fori_loop(0, D, dstep, accs0)
    for g in range(GROUPS):
        score_v[pl.ds(g * LANES, LANES)] = accs[g]

    pltpu.sync_copy(score_v, out_hbm.at[pl.ds(base, BPW)])


@jax.jit
def _svd_score(user_ids, item_ids, user_emb, item_emb, ubias_t, ibias_t):
    mesh = plsc.VectorSubcoreMesh(core_axis_name="c", subcore_axis_name="s")
    run = functools.partial(
        pl.kernel,
        out_type=jax.ShapeDtypeStruct((B,), jnp.float32),
        mesh=mesh,
        compiler_params=pltpu.CompilerParams(
            needs_layout_passes=False, use_tc_tiling_on_sc=False),
        scratch_types=[
            pltpu.VMEM((BPW,), jnp.int32),
            pltpu.VMEM((BPW,), jnp.int32),
            pltpu.VMEM((BPW, D), jnp.float32),
            pltpu.VMEM((BPW, D), jnp.float32),
            pltpu.VMEM((BPW,), jnp.float32),
            pltpu.VMEM((BPW,), jnp.float32),
            pltpu.VMEM((BPW,), jnp.float32),
            pltpu.SemaphoreType.DMA,
        ],
    )(_svd_body)
    return run(user_ids, item_ids, user_emb, item_emb, ubias_t, ibias_t)


def kernel(user_ids, item_ids, user_emb, item_emb, user_bias, item_bias,
           average_score):
    del average_score
    score = _svd_score(user_ids, item_ids, user_emb, item_emb,
                       user_bias.T, item_bias.T)
    return score.reshape(B, 1)
